# Initial kernel scaffold; baseline (speedup 1.0000x reference)
import jax
import jax.numpy as jnp
from jax.experimental import pallas as pl


def kernel(coords, coord_mask, res_idx, padding_mask, confidence, params):
    raise NotImplementedError("write your pallas kernel here")


# speedup vs baseline: 11.1049x; 11.1049x over previous
import jax
import jax.numpy as jnp
import numpy as np
from jax import lax
from jax.experimental import pallas as pl

EPS = 1e-4
K = 16


def _norm(x, axis=-1, keepdims=False):
    return jnp.sqrt(jnp.sum(x * x, axis=axis, keepdims=keepdims) + 1e-8)


def _normalize(x, axis=-1):
    return x / _norm(x, axis=axis, keepdims=True)


def _dihedrals(X):
    Xf = X.reshape(X.shape[0], -1, 3)
    dX = Xf[:, 1:] - Xf[:, :-1]
    U = _normalize(dX)
    u2, u1, u0 = U[:, :-2], U[:, 1:-1], U[:, 2:]
    n2 = _normalize(jnp.cross(u2, u1))
    n1 = _normalize(jnp.cross(u1, u0))
    cosD = jnp.clip(jnp.sum(n2 * n1, axis=-1), -1 + 1e-6, 1 - 1e-6)
    D = jnp.sign(jnp.sum(u2 * n1, axis=-1)) * jnp.arccos(cosD)
    D = jnp.pad(D, ((0, 0), (1, 2)))
    D = D.reshape(X.shape[0], -1, 3)
    return jnp.concatenate([jnp.cos(D), jnp.sin(D)], axis=-1)


def _orientations(Xca):
    f = _normalize(Xca[:, 1:] - Xca[:, :-1])
    b = _normalize(Xca[:, :-1] - Xca[:, 1:])
    f = jnp.pad(f, ((0, 0), (0, 1), (0, 0)))
    b = jnp.pad(b, ((0, 0), (1, 0), (0, 0)))
    return jnp.stack([f, b], axis=-2)


def _sidechains(X):
    n, ca, c = X[:, :, 0], X[:, :, 1], X[:, :, 2]
    u = _normalize(c - ca)
    v = _normalize(n - ca)
    bis = _normalize(u + v)
    perp = _normalize(jnp.cross(u, v))
    vec = -bis * (1.0 / np.sqrt(3.0)) - perp * np.sqrt(2.0 / 3.0)
    return vec[:, :, None, :]


def _rbf(d, n_bins=16, d_max=20.0):
    mu = jnp.linspace(0.0, d_max, n_bins)
    sigma = d_max / n_bins
    return jnp.exp(-(((d[..., None] - mu) / sigma) ** 2))


def _pos_emb(d_rel, num=16):
    freq = jnp.exp(jnp.arange(0, num, 2, dtype=jnp.float32) * (-np.log(1000.0) / num))
    ang = d_rel[..., None].astype(jnp.float32) * freq
    return jnp.concatenate([jnp.cos(ang), jnp.sin(ang)], axis=-1)


def _gvp_jax(p, s, V, act=True):
    Vh = jnp.einsum("nvc,vh->nhc", V, p["Wh"])
    vn = _norm(Vh, axis=-1)
    s_out = jnp.concatenate([s, vn], axis=-1) @ p["Ws"] + p["bs"]
    Vu = jnp.einsum("nhc,hu->nuc", Vh, p["Wv"])
    gate = jax.nn.sigmoid(s_out @ p["Wg"] + p["bg"])
    V_out = Vu * gate[..., None]
    if act:
        s_out = jax.nn.relu(s_out)
    return s_out, V_out


def _ln_jax(s, V):
    mu = jnp.mean(s, axis=-1, keepdims=True)
    var = jnp.var(s, axis=-1, keepdims=True)
    s = (s - mu) / jnp.sqrt(var + EPS)
    vn2 = jnp.sum(V * V, axis=-1)
    rms = jnp.sqrt(jnp.mean(vn2, axis=-1, keepdims=True) + EPS)
    V = V / rms[..., None]
    return s, V


def _sigmoid(z):
    e = jnp.exp(-jnp.abs(z))
    return jnp.where(z >= 0, 1.0 / (1.0 + e), e / (1.0 + e))


def _conv_layer_kernel(L, BLKD, ns, nv):
    BLKE = BLKD * K

    def gvp(p, s, vx, vy, vz, act):
        Wh, Ws, bs, Wv, Wg, bg = p
        w = Wh[...]
        hx = jnp.dot(vx, w, preferred_element_type=jnp.float32)
        hy = jnp.dot(vy, w, preferred_element_type=jnp.float32)
        hz = jnp.dot(vz, w, preferred_element_type=jnp.float32)
        vn = jnp.sqrt(hx * hx + hy * hy + hz * hz + 1e-8)
        so = jnp.dot(jnp.concatenate([s, vn], axis=-1), Ws[...],
                     preferred_element_type=jnp.float32) + bs[...]
        wv = Wv[...]
        ux = jnp.dot(hx, wv, preferred_element_type=jnp.float32)
        uy = jnp.dot(hy, wv, preferred_element_type=jnp.float32)
        uz = jnp.dot(hz, wv, preferred_element_type=jnp.float32)
        gate = _sigmoid(jnp.dot(so, Wg[...], preferred_element_type=jnp.float32)
                        + bg[...])
        if act:
            so = jnp.maximum(so, 0.0)
        return so, ux * gate, uy * gate, uz * gate

    def ln(s, vx, vy, vz):
        mu = jnp.mean(s, axis=-1, keepdims=True)
        var = jnp.mean((s - mu) * (s - mu), axis=-1, keepdims=True)
        s = (s - mu) / jnp.sqrt(var + EPS)
        vn2 = vx * vx + vy * vy + vz * vz
        rms = jnp.sqrt(jnp.mean(vn2, axis=-1, keepdims=True) + EPS)
        inv = 1.0 / rms
        return s, vx * inv, vy * inv, vz * inv

    def body(src_ref, s_ref, V_ref, se_ref, Ve_ref, *refs):
        s_out_ref, V_out_ref = refs[-2:]
        prefs = refs[:-2]
        p = [prefs[6 * j:6 * j + 6] for j in range(5)]
        i = pl.program_id(1)
        off = i * BLKD

        idx = src_ref[0, 0]
        cols = lax.broadcasted_iota(jnp.int32, (BLKE, L), 1)
        onehot = (idx == cols).astype(jnp.float32)
        s_all = s_ref[0]
        g_s = jnp.dot(onehot, s_all, preferred_element_type=jnp.float32)
        g_vx = jnp.dot(onehot, V_ref[0, 0], preferred_element_type=jnp.float32)
        g_vy = jnp.dot(onehot, V_ref[0, 1], preferred_element_type=jnp.float32)
        g_vz = jnp.dot(onehot, V_ref[0, 2], preferred_element_type=jnp.float32)

        s_d = s_ref[0, pl.ds(off, BLKD), :]
        vx_d = V_ref[0, 0, pl.ds(off, BLKD), :]
        vy_d = V_ref[0, 1, pl.ds(off, BLKD), :]
        vz_d = V_ref[0, 2, pl.ds(off, BLKD), :]

        erow = lax.broadcasted_iota(jnp.int32, (BLKE, BLKD), 0)
        dcol = lax.broadcasted_iota(jnp.int32, (BLKE, BLKD), 1)
        e2d = ((erow // K) == dcol).astype(jnp.float32)
        drow = lax.broadcasted_iota(jnp.int32, (BLKD, BLKE), 0)
        ecol = lax.broadcasted_iota(jnp.int32, (BLKD, BLKE), 1)
        d2e = (drow == (ecol // K)).astype(jnp.float32)

        def expand(a):
            return jnp.dot(e2d, a, preferred_element_type=jnp.float32)

        se_b = se_ref[0]
        ve = Ve_ref[0]
        ms = jnp.concatenate([expand(s_d), se_b, g_s], axis=-1)
        mvx = jnp.concatenate([expand(vx_d), ve[:, 0:1], g_vx], axis=-1)
        mvy = jnp.concatenate([expand(vy_d), ve[:, 1:2], g_vy], axis=-1)
        mvz = jnp.concatenate([expand(vz_d), ve[:, 2:3], g_vz], axis=-1)

        ms, mvx, mvy, mvz = gvp(p[0], ms, mvx, mvy, mvz, True)
        ms, mvx, mvy, mvz = gvp(p[1], ms, mvx, mvy, mvz, True)
        ms, mvx, mvy, mvz = gvp(p[2], ms, mvx, mvy, mvz, False)

        inv_k = np.float32(1.0 / K)

        def agg(m):
            return jnp.dot(d2e, m, preferred_element_type=jnp.float32) * inv_k

        s1, vx1, vy1, vz1 = ln(s_d + agg(ms), vx_d + agg(mvx),
                               vy_d + agg(mvy), vz_d + agg(mvz))
        hs, hvx, hvy, hvz = gvp(p[3], s1, vx1, vy1, vz1, True)
        hs, hvx, hvy, hvz = gvp(p[4], hs, hvx, hvy, hvz, False)
        s2, vx2, vy2, vz2 = ln(s1 + hs, vx1 + hvx, vy1 + hvy, vz1 + hvz)

        s_out_ref[0] = s2
        V_out_ref[0, 0] = vx2
        V_out_ref[0, 1] = vy2
        V_out_ref[0, 2] = vz2

    return body


def _conv_layer(sN, Vp, seP, VeP, srcP, lp):
    Bc, L, ns = sN.shape
    nv = Vp.shape[-1]
    es = seP.shape[-1]
    BLKD = 128
    NBLK = L // BLKD
    BLKE = BLKD * K

    params = []
    pspecs = []
    for name in ("msg0", "msg1", "msg2", "ff0", "ff1"):
        g = lp[name]
        for w in (g["Wh"], g["Ws"], g["bs"].reshape(1, -1),
                  g["Wv"], g["Wg"], g["bg"].reshape(1, -1)):
            params.append(w)
            pspecs.append(pl.BlockSpec(w.shape, lambda b, i: (0, 0)))

    body = _conv_layer_kernel(L, BLKD, ns, nv)
    out = pl.pallas_call(
        body,
        grid=(Bc, NBLK),
        in_specs=[
            pl.BlockSpec((1, 1, BLKE, 1), lambda b, i: (b, i, 0, 0)),
            pl.BlockSpec((1, L, ns), lambda b, i: (b, 0, 0)),
            pl.BlockSpec((1, 3, L, nv), lambda b, i: (b, 0, 0, 0)),
            pl.BlockSpec((1, BLKE, es), lambda b, i: (b, i, 0)),
            pl.BlockSpec((1, BLKE, 3), lambda b, i: (b, i, 0)),
        ] + pspecs,
        out_specs=[
            pl.BlockSpec((1, BLKD, ns), lambda b, i: (b, i, 0)),
            pl.BlockSpec((1, 3, BLKD, nv), lambda b, i: (b, 0, i, 0)),
        ],
        out_shape=[
            jax.ShapeDtypeStruct((Bc, L, ns), jnp.float32),
            jax.ShapeDtypeStruct((Bc, 3, L, nv), jnp.float32),
        ],
    )(srcP, sN, Vp, seP, VeP, *params)
    return out[0], out[1]


def kernel(coords, coord_mask, res_idx, padding_mask, confidence, params):
    Bc, L = coords.shape[0], coords.shape[1]
    N = Bc * L
    Xca = coords[:, :, 1]

    D2 = jnp.sum((Xca[:, :, None] - Xca[:, None, :]) ** 2, axis=-1)
    pad = padding_mask.astype(jnp.float32)
    D2 = D2 + 1e8 * pad[:, None, :] + 1e8 * pad[:, :, None]
    _, nbr = lax.top_k(-D2, K)
    offs = (jnp.arange(Bc) * L)[:, None, None]
    src = (nbr + offs).reshape(-1)
    dst = jnp.repeat(jnp.arange(N), K)

    s_n = jnp.concatenate([_dihedrals(coords), confidence[..., None]],
                          axis=-1).reshape(N, -1)
    V_n = jnp.concatenate([_orientations(Xca), _sidechains(coords)],
                          axis=-2).reshape(N, 3, 3)
    Xf = Xca.reshape(N, 3)
    dvec = Xf[src] - Xf[dst]
    dist = _norm(dvec)
    ridx = res_idx.reshape(N)
    drel = jnp.clip(ridx[src] - ridx[dst], -32, 32)
    s_e = jnp.concatenate([_rbf(dist), _pos_emb(drel)], axis=-1)
    V_e = _normalize(dvec)[:, None, :]

    s, V = _gvp_jax(params["node_embed"], s_n, V_n)
    s, V = _ln_jax(s, V)
    se, Ve = _gvp_jax(params["edge_embed"], s_e, V_e)
    se, Ve = _ln_jax(se, Ve)

    ns, nv = s.shape[-1], V.shape[-2]
    es = se.shape[-1]

    sN = s.reshape(Bc, L, ns)
    Vp = V.reshape(Bc, L, nv, 3).transpose(0, 3, 1, 2)
    seP = se.reshape(Bc, L * K, es)
    VeP = Ve.reshape(Bc, L * K, 3)
    BLKD = 128
    srcP = nbr.reshape(Bc, L // BLKD, BLKD * K, 1).astype(jnp.int32)

    for lp in params["layers"]:
        sN, Vp = _conv_layer(sN, Vp, seP, VeP, srcP, lp)

    s_out = sN
    V_out = Vp.transpose(0, 2, 3, 1)
    return s_out, V_out


# speedup vs baseline: 15.5968x; 1.4045x over previous
import functools

import jax
import jax.numpy as jnp
import numpy as np
from jax import lax
from jax.experimental import pallas as pl

EPS = 1e-4
K = 16
NS, NV = 100, 16
ES = 32
BLKD = 128
BLKE = BLKD * K


def _norm(x, axis=-1, keepdims=False):
    return jnp.sqrt(jnp.sum(x * x, axis=axis, keepdims=keepdims) + 1e-8)


def _normalize(x, axis=-1):
    return x / _norm(x, axis=axis, keepdims=True)


def _dihedrals(X):
    Xf = X.reshape(X.shape[0], -1, 3)
    dX = Xf[:, 1:] - Xf[:, :-1]
    U = _normalize(dX)
    u2, u1, u0 = U[:, :-2], U[:, 1:-1], U[:, 2:]
    n2 = _normalize(jnp.cross(u2, u1))
    n1 = _normalize(jnp.cross(u1, u0))
    cosD = jnp.clip(jnp.sum(n2 * n1, axis=-1), -1 + 1e-6, 1 - 1e-6)
    D = jnp.sign(jnp.sum(u2 * n1, axis=-1)) * jnp.arccos(cosD)
    D = jnp.pad(D, ((0, 0), (1, 2)))
    D = D.reshape(X.shape[0], -1, 3)
    return jnp.concatenate([jnp.cos(D), jnp.sin(D)], axis=-1)


def _orientations(Xca):
    f = _normalize(Xca[:, 1:] - Xca[:, :-1])
    b = _normalize(Xca[:, :-1] - Xca[:, 1:])
    f = jnp.pad(f, ((0, 0), (0, 1), (0, 0)))
    b = jnp.pad(b, ((0, 0), (1, 0), (0, 0)))
    return jnp.stack([f, b], axis=-2)


def _sidechains(X):
    n, ca, c = X[:, :, 0], X[:, :, 1], X[:, :, 2]
    u = _normalize(c - ca)
    v = _normalize(n - ca)
    bis = _normalize(u + v)
    perp = _normalize(jnp.cross(u, v))
    vec = -bis * (1.0 / np.sqrt(3.0)) - perp * np.sqrt(2.0 / 3.0)
    return vec[:, :, None, :]


def _sigmoid(z):
    e = jnp.exp(-jnp.abs(z))
    return jnp.where(z >= 0, 1.0 / (1.0 + e), e / (1.0 + e))


def _dot(a, b):
    return jnp.dot(a, b, preferred_element_type=jnp.float32)


def _gdot(onehot_bf16, x, terms=2):
    acc = None
    r = x
    for _ in range(terms):
        h = r.astype(jnp.bfloat16)
        p = jnp.dot(onehot_bf16, h, preferred_element_type=jnp.float32)
        acc = p if acc is None else acc + p
        r = r - h.astype(jnp.float32)
    return acc


def _gvp_cm(p, s, vx, vy, vz, act):
    Wh, Ws, bs, Wv, Wg, bg = p
    w = Wh[...]
    hx, hy, hz = _dot(vx, w), _dot(vy, w), _dot(vz, w)
    vn = jnp.sqrt(hx * hx + hy * hy + hz * hz + 1e-8)
    so = _dot(jnp.concatenate([s, vn], axis=-1), Ws[...]) + bs[...]
    wv = Wv[...]
    ux, uy, uz = _dot(hx, wv), _dot(hy, wv), _dot(hz, wv)
    gate = _sigmoid(_dot(so, Wg[...]) + bg[...])
    if act:
        so = jnp.maximum(so, 0.0)
    return so, ux * gate, uy * gate, uz * gate


def _ln_cm(s, vx, vy, vz):
    mu = jnp.mean(s, axis=-1, keepdims=True)
    var = jnp.mean((s - mu) * (s - mu), axis=-1, keepdims=True)
    s = (s - mu) / jnp.sqrt(var + EPS)
    vn2 = vx * vx + vy * vy + vz * vz
    inv = 1.0 / jnp.sqrt(jnp.mean(vn2, axis=-1, keepdims=True) + EPS)
    return s, vx * inv, vy * inv, vz * inv


def _onehot_of(idx_ref, L):
    idx = idx_ref[0, 0]
    cols = lax.broadcasted_iota(jnp.int32, (1, L), 1)
    return (idx == cols).astype(jnp.bfloat16)


def _expand(a):
    return jnp.broadcast_to(a[:, None, :],
                            (BLKD, K, a.shape[-1])).reshape(BLKE, a.shape[-1])


def _edge_kernel(L):
    def body(src_ref, xr_ref, pt_ref, Wh, Ws, bs, Wv, Wg, bg,
             se_out, ve_out):
        i = pl.program_id(1)
        off = i * BLKD
        onehot = _onehot_of(src_ref, L)

        g4 = _gdot(onehot, xr_ref[0], terms=3)
        xr_d = _expand(xr_ref[0, pl.ds(off, BLKD), :])
        gx, gr = g4[:, 0:3], g4[:, 3:4]
        xd, rd = xr_d[:, 0:3], xr_d[:, 3:4]

        dvec = gx - xd
        d2 = jnp.sum(dvec * dvec, axis=-1, keepdims=True)
        dist = jnp.sqrt(d2 + 1e-8)
        ve = dvec / jnp.sqrt(d2 + 1e-8)

        mu = (lax.broadcasted_iota(jnp.int32, (1, 16), 1).astype(jnp.float32)
              * np.float32(20.0 / 15.0))
        z = (dist - mu) * np.float32(16.0 / 20.0)
        rbf = jnp.exp(-(z * z))
        drel = jnp.clip(gr - rd, -32.0, 32.0).astype(jnp.int32) + 32
        tcols = lax.broadcasted_iota(jnp.int32, (1, 65), 1)
        oh65 = (drel == tcols).astype(jnp.bfloat16)
        pemb = _gdot(oh65, pt_ref[...], terms=2)
        s_e = jnp.concatenate([rbf, pemb], axis=-1)

        vex, vey, vez = ve[:, 0:1], ve[:, 1:2], ve[:, 2:3]
        w = Wh[...]
        hx, hy, hz = vex * w, vey * w, vez * w
        vn = jnp.sqrt(hx * hx + hy * hy + hz * hz + 1e-8)
        so = _dot(jnp.concatenate([s_e, vn], axis=-1), Ws[...]) + bs[...]
        wv = Wv[...]
        gate = _sigmoid(_dot(so, Wg[...]) + bg[...])
        ux, uy, uz = hx * wv * gate, hy * wv * gate, hz * wv * gate
        so = jnp.maximum(so, 0.0)
        so, ux, uy, uz = _ln_cm(so, ux, uy, uz)
        se_out[0] = so
        ve_out[0] = jnp.concatenate([ux, uy, uz], axis=-1)

    return body


def _pos_emb_table(num=16):
    d = jnp.arange(-32, 33, dtype=jnp.float32)
    freq = jnp.exp(jnp.arange(0, num, 2, dtype=jnp.float32)
                   * (-np.log(1000.0) / num))
    ang = d[:, None] * freq
    return jnp.concatenate([jnp.cos(ang), jnp.sin(ang)], axis=-1)


def _edge_features(XcaP, ridxP, srcP, ep):
    Bc, L, _ = XcaP.shape
    NBLK = L // BLKD
    xr = jnp.concatenate([XcaP, ridxP], axis=-1)
    ptab = _pos_emb_table()
    params = [ep["Wh"], ep["Ws"], ep["bs"].reshape(1, -1),
              ep["Wv"], ep["Wg"], ep["bg"].reshape(1, -1)]
    pspecs = [pl.BlockSpec(w.shape, lambda b, i: (0, 0)) for w in params]
    out = pl.pallas_call(
        _edge_kernel(L),
        grid=(Bc, NBLK),
        in_specs=[
            pl.BlockSpec((1, 1, BLKE, 1), lambda b, i: (b, i, 0, 0)),
            pl.BlockSpec((1, L, 4), lambda b, i: (b, 0, 0)),
            pl.BlockSpec(ptab.shape, lambda b, i: (0, 0)),
        ] + pspecs,
        out_specs=[
            pl.BlockSpec((1, BLKE, ES), lambda b, i: (b, i, 0)),
            pl.BlockSpec((1, BLKE, 3), lambda b, i: (b, i, 0)),
        ],
        out_shape=[
            jax.ShapeDtypeStruct((Bc, L * K, ES), jnp.float32),
            jax.ShapeDtypeStruct((Bc, L * K, 3), jnp.float32),
        ],
    )(srcP, xr, ptab, *params)
    return out[0], out[1]


def _node_kernel(BLKN):
    def body(s_ref, v_ref, Wh, Ws, bs, Wv, Wg, bg, out_ref):
        s = s_ref[0]
        vx, vy, vz = v_ref[0, 0], v_ref[0, 1], v_ref[0, 2]
        p = (Wh, Ws, bs, Wv, Wg, bg)
        so, ux, uy, uz = _gvp_cm(p, s, vx, vy, vz, True)
        so, ux, uy, uz = _ln_cm(so, ux, uy, uz)
        out_ref[0] = jnp.concatenate([so, ux, uy, uz], axis=-1)

    return body


def _node_embed(s_n, V_cm, npar):
    Bc, L, si = s_n.shape
    BLKN = min(512, L)
    NBLK = L // BLKN
    NF = NS + 3 * NV
    params = [npar["Wh"], npar["Ws"], npar["bs"].reshape(1, -1),
              npar["Wv"], npar["Wg"], npar["bg"].reshape(1, -1)]
    pspecs = [pl.BlockSpec(w.shape, lambda b, i: (0, 0)) for w in params]
    out = pl.pallas_call(
        _node_kernel(BLKN),
        grid=(Bc, NBLK),
        in_specs=[
            pl.BlockSpec((1, BLKN, si), lambda b, i: (b, i, 0)),
            pl.BlockSpec((1, 3, BLKN, 3), lambda b, i: (b, 0, i, 0)),
        ] + pspecs,
        out_specs=pl.BlockSpec((1, BLKN, NF), lambda b, i: (b, i, 0)),
        out_shape=jax.ShapeDtypeStruct((Bc, L, NF), jnp.float32),
    )(s_n, V_cm, *params)
    return out


def _conv_layer_kernel(L):
    NF = NS + 3 * NV

    def body(src_ref, nf_ref, se_ref, ve_ref, *refs):
        out_ref = refs[-1]
        prefs = refs[:-1]
        p = [prefs[6 * j:6 * j + 6] for j in range(5)]
        i = pl.program_id(1)
        off = i * BLKD

        onehot = _onehot_of(src_ref, L)

        g = _gdot(onehot, nf_ref[0], terms=2)
        nf_d = nf_ref[0, pl.ds(off, BLKD), :]
        ex = _expand(nf_d)

        se_b = se_ref[0]
        ve = ve_ref[0]
        ms = jnp.concatenate([ex[:, :NS], se_b, g[:, :NS]], axis=-1)
        c0, c1, c2, c3 = NS, NS + NV, NS + 2 * NV, NF
        mvx = jnp.concatenate([ex[:, c0:c1], ve[:, 0:1], g[:, c0:c1]], axis=-1)
        mvy = jnp.concatenate([ex[:, c1:c2], ve[:, 1:2], g[:, c1:c2]], axis=-1)
        mvz = jnp.concatenate([ex[:, c2:c3], ve[:, 2:3], g[:, c2:c3]], axis=-1)

        ms, mvx, mvy, mvz = _gvp_cm(p[0], ms, mvx, mvy, mvz, True)
        ms, mvx, mvy, mvz = _gvp_cm(p[1], ms, mvx, mvy, mvz, True)
        ms, mvx, mvy, mvz = _gvp_cm(p[2], ms, mvx, mvy, mvz, False)

        packed_m = jnp.concatenate([ms, mvx, mvy, mvz], axis=-1)
        aggp = jnp.sum(packed_m.reshape(BLKD, K, NF),
                       axis=1) * np.float32(1.0 / K)
        new = nf_d + aggp

        s1, vx1, vy1, vz1 = _ln_cm(new[:, :NS], new[:, c0:c1],
                                   new[:, c1:c2], new[:, c2:c3])
        hs, hvx, hvy, hvz = _gvp_cm(p[3], s1, vx1, vy1, vz1, True)
        hs, hvx, hvy, hvz = _gvp_cm(p[4], hs, hvx, hvy, hvz, False)
        s2, vx2, vy2, vz2 = _ln_cm(s1 + hs, vx1 + hvx, vy1 + hvy, vz1 + hvz)

        out_ref[0] = jnp.concatenate([s2, vx2, vy2, vz2], axis=-1)

    return body


def _conv_layer(nf, seP, veP, srcP, lp):
    Bc, L, NF = nf.shape
    NBLK = L // BLKD
    params = []
    for name in ("msg0", "msg1", "msg2", "ff0", "ff1"):
        g = lp[name]
        params += [g["Wh"], g["Ws"], g["bs"].reshape(1, -1),
                   g["Wv"], g["Wg"], g["bg"].reshape(1, -1)]
    pspecs = [pl.BlockSpec(w.shape, lambda b, i: (0, 0)) for w in params]
    out = pl.pallas_call(
        _conv_layer_kernel(L),
        grid=(Bc, NBLK),
        in_specs=[
            pl.BlockSpec((1, 1, BLKE, 1), lambda b, i: (b, i, 0, 0)),
            pl.BlockSpec((1, L, NF), lambda b, i: (b, 0, 0)),
            pl.BlockSpec((1, BLKE, ES), lambda b, i: (b, i, 0)),
            pl.BlockSpec((1, BLKE, 3), lambda b, i: (b, i, 0)),
        ] + pspecs,
        out_specs=pl.BlockSpec((1, BLKD, NF), lambda b, i: (b, i, 0)),
        out_shape=jax.ShapeDtypeStruct((Bc, L, NF), jnp.float32),
    )(srcP, nf, seP, veP, *params)
    return out


def kernel(coords, coord_mask, res_idx, padding_mask, confidence, params):
    Bc, L = coords.shape[0], coords.shape[1]
    Xca = coords[:, :, 1]

    D2 = jnp.sum((Xca[:, :, None] - Xca[:, None, :]) ** 2, axis=-1)
    pad = padding_mask.astype(jnp.float32)
    D2 = D2 + 1e8 * pad[:, None, :] + 1e8 * pad[:, :, None]
    _, nbr = lax.top_k(-D2, K)
    srcP = nbr.reshape(Bc, L // BLKD, BLKD * K, 1).astype(jnp.int32)

    ridxP = res_idx.astype(jnp.float32)[..., None]
    seP, veP = _edge_features(Xca, ridxP, srcP, params["edge_embed"])

    s_n = jnp.concatenate([_dihedrals(coords), confidence[..., None]], axis=-1)
    V_n = jnp.concatenate([_orientations(Xca), _sidechains(coords)], axis=-2)
    V_cm = V_n.reshape(Bc, L, 3, 3).transpose(0, 3, 1, 2)
    nf = _node_embed(s_n, V_cm, params["node_embed"])

    for lp in params["layers"]:
        nf = _conv_layer(nf, seP, veP, srcP, lp)

    s_out = nf[:, :, :NS]
    V_out = nf[:, :, NS:].reshape(Bc, L, 3, NV).transpose(0, 1, 3, 2)
    return s_out, V_out


# speedup vs baseline: 18.8945x; 1.2114x over previous
import functools

import jax
import jax.numpy as jnp
import numpy as np
from jax import lax
from jax.experimental import pallas as pl

EPS = 1e-4
K = 16
NS, NV = 100, 16
ES = 32
BLKD = 128
BLKE = BLKD * K


def _norm(x, axis=-1, keepdims=False):
    return jnp.sqrt(jnp.sum(x * x, axis=axis, keepdims=keepdims) + 1e-8)


def _normalize(x, axis=-1):
    return x / _norm(x, axis=axis, keepdims=True)


def _dihedrals(X):
    Xf = X.reshape(X.shape[0], -1, 3)
    dX = Xf[:, 1:] - Xf[:, :-1]
    U = _normalize(dX)
    u2, u1, u0 = U[:, :-2], U[:, 1:-1], U[:, 2:]
    n2 = _normalize(jnp.cross(u2, u1))
    n1 = _normalize(jnp.cross(u1, u0))
    cosD = jnp.clip(jnp.sum(n2 * n1, axis=-1), -1 + 1e-6, 1 - 1e-6)
    D = jnp.sign(jnp.sum(u2 * n1, axis=-1)) * jnp.arccos(cosD)
    D = jnp.pad(D, ((0, 0), (1, 2)))
    D = D.reshape(X.shape[0], -1, 3)
    return jnp.concatenate([jnp.cos(D), jnp.sin(D)], axis=-1)


def _orientations(Xca):
    f = _normalize(Xca[:, 1:] - Xca[:, :-1])
    b = _normalize(Xca[:, :-1] - Xca[:, 1:])
    f = jnp.pad(f, ((0, 0), (0, 1), (0, 0)))
    b = jnp.pad(b, ((0, 0), (1, 0), (0, 0)))
    return jnp.stack([f, b], axis=-2)


def _sidechains(X):
    n, ca, c = X[:, :, 0], X[:, :, 1], X[:, :, 2]
    u = _normalize(c - ca)
    v = _normalize(n - ca)
    bis = _normalize(u + v)
    perp = _normalize(jnp.cross(u, v))
    vec = -bis * (1.0 / np.sqrt(3.0)) - perp * np.sqrt(2.0 / 3.0)
    return vec[:, :, None, :]


def _sigmoid(z):
    e = jnp.exp(-jnp.abs(z))
    return jnp.where(z >= 0, 1.0 / (1.0 + e), e / (1.0 + e))


def _dot(a, b):
    return jnp.dot(a, b, preferred_element_type=jnp.float32)


def _gdot(onehot_bf16, x, terms=2):
    acc = None
    r = x
    for _ in range(terms):
        h = r.astype(jnp.bfloat16)
        p = jnp.dot(onehot_bf16, h, preferred_element_type=jnp.float32)
        acc = p if acc is None else acc + p
        r = r - h.astype(jnp.float32)
    return acc


def _gvp_cm(p, s, vx, vy, vz, act):
    Wh, Ws, bs, Wv, Wg, bg = p
    w = Wh[...]
    hx, hy, hz = _dot(vx, w), _dot(vy, w), _dot(vz, w)
    vn = jnp.sqrt(hx * hx + hy * hy + hz * hz + 1e-8)
    so = _dot(jnp.concatenate([s, vn], axis=-1), Ws[...]) + bs[...]
    wv = Wv[...]
    ux, uy, uz = _dot(hx, wv), _dot(hy, wv), _dot(hz, wv)
    gate = _sigmoid(_dot(so, Wg[...]) + bg[...])
    if act:
        so = jnp.maximum(so, 0.0)
    return so, ux * gate, uy * gate, uz * gate


def _ln_cm(s, vx, vy, vz):
    mu = jnp.mean(s, axis=-1, keepdims=True)
    var = jnp.mean((s - mu) * (s - mu), axis=-1, keepdims=True)
    s = (s - mu) / jnp.sqrt(var + EPS)
    vn2 = vx * vx + vy * vy + vz * vz
    inv = 1.0 / jnp.sqrt(jnp.mean(vn2, axis=-1, keepdims=True) + EPS)
    return s, vx * inv, vy * inv, vz * inv


def _onehot_of(idx_ref, L):
    idx = idx_ref[0, 0]
    cols = lax.broadcasted_iota(jnp.int32, (1, L), 1)
    return (idx == cols).astype(jnp.bfloat16)


def _expand(a):
    return jnp.broadcast_to(a[:, None, :],
                            (BLKD, K, a.shape[-1])).reshape(BLKE, a.shape[-1])


def _knn_kernel(L):
    def body(x_ref, xT_ref, pad_ref, padT_ref, out_ref, keys_scr, m_scr):
        k = pl.program_id(1)

        @pl.when(k == 0)
        def _init():
            x = x_ref[0]
            xT = xT_ref[0]
            d0 = x[:, 0:1] - xT[0:1, :]
            d1 = x[:, 1:2] - xT[1:2, :]
            d2c = x[:, 2:3] - xT[2:3, :]
            D2 = d0 * d0 + d1 * d1 + d2c * d2c
            D2 = D2 + pad_ref[0] * 1e8 + padT_ref[0] * 1e8
            u = lax.bitcast_convert_type(D2, jnp.int32)
            lanes = lax.broadcasted_iota(jnp.int32, (1, L), 1)
            keys_scr[...] = (u & jnp.int32(-1024)) | lanes
            m_scr[...] = jnp.full((L, 1), jnp.int32(-2147483648))

        keys = keys_scr[...]
        mprev = m_scr[...]
        cand = jnp.where(keys > mprev, keys, jnp.int32(2147483647))
        m = jnp.min(cand, axis=-1, keepdims=True)
        m_scr[...] = m
        out_ref[0, 0] = m & jnp.int32(1023)

    return body


def _knn(Xca, padding_mask):
    from jax.experimental.pallas import tpu as pltpu
    Bc, L, _ = Xca.shape
    XcaT = Xca.transpose(0, 2, 1)
    padf = padding_mask.astype(jnp.float32)
    pad_c = padf[:, :, None]
    pad_r = padf[:, None, :]
    out = pl.pallas_call(
        _knn_kernel(L),
        grid=(Bc, K),
        in_specs=[
            pl.BlockSpec((1, L, 3), lambda b, k: (b, 0, 0)),
            pl.BlockSpec((1, 3, L), lambda b, k: (b, 0, 0)),
            pl.BlockSpec((1, L, 1), lambda b, k: (b, 0, 0)),
            pl.BlockSpec((1, 1, L), lambda b, k: (b, 0, 0)),
        ],
        out_specs=pl.BlockSpec((1, 1, L, 1), lambda b, k: (b, k, 0, 0)),
        out_shape=jax.ShapeDtypeStruct((Bc, K, L, 1), jnp.int32),
        scratch_shapes=[pltpu.VMEM((L, L), jnp.int32),
                        pltpu.VMEM((L, 1), jnp.int32)],
    )(Xca, XcaT, pad_c, pad_r)
    return out.reshape(Bc, K, L).transpose(0, 2, 1)


def _edge_kernel(L):
    def body(src_ref, xr_ref, pt_ref, Wh, Ws, bs, Wv, Wg, bg,
             se_out, ve_out):
        i = pl.program_id(1)
        off = i * BLKD
        onehot = _onehot_of(src_ref, L)

        g4 = _gdot(onehot, xr_ref[0], terms=3)
        xr_d = _expand(xr_ref[0, pl.ds(off, BLKD), :])
        gx, gr = g4[:, 0:3], g4[:, 3:4]
        xd, rd = xr_d[:, 0:3], xr_d[:, 3:4]

        dvec = gx - xd
        d2 = jnp.sum(dvec * dvec, axis=-1, keepdims=True)
        dist = jnp.sqrt(d2 + 1e-8)
        ve = dvec / jnp.sqrt(d2 + 1e-8)

        mu = (lax.broadcasted_iota(jnp.int32, (1, 16), 1).astype(jnp.float32)
              * np.float32(20.0 / 15.0))
        z = (dist - mu) * np.float32(16.0 / 20.0)
        rbf = jnp.exp(-(z * z))
        drel = jnp.clip(gr - rd, -32.0, 32.0).astype(jnp.int32) + 32
        tcols = lax.broadcasted_iota(jnp.int32, (1, 65), 1)
        oh65 = (drel == tcols).astype(jnp.bfloat16)
        pemb = _gdot(oh65, pt_ref[...], terms=2)
        s_e = jnp.concatenate([rbf, pemb], axis=-1)

        vex, vey, vez = ve[:, 0:1], ve[:, 1:2], ve[:, 2:3]
        w = Wh[...]
        hx, hy, hz = vex * w, vey * w, vez * w
        vn = jnp.sqrt(hx * hx + hy * hy + hz * hz + 1e-8)
        so = _dot(jnp.concatenate([s_e, vn], axis=-1), Ws[...]) + bs[...]
        wv = Wv[...]
        gate = _sigmoid(_dot(so, Wg[...]) + bg[...])
        ux, uy, uz = hx * wv * gate, hy * wv * gate, hz * wv * gate
        so = jnp.maximum(so, 0.0)
        so, ux, uy, uz = _ln_cm(so, ux, uy, uz)
        se_out[0] = so
        ve_out[0] = jnp.concatenate([ux, uy, uz], axis=-1)

    return body


def _pos_emb_table(num=16):
    d = jnp.arange(-32, 33, dtype=jnp.float32)
    freq = jnp.exp(jnp.arange(0, num, 2, dtype=jnp.float32)
                   * (-np.log(1000.0) / num))
    ang = d[:, None] * freq
    return jnp.concatenate([jnp.cos(ang), jnp.sin(ang)], axis=-1)


def _edge_features(XcaP, ridxP, srcP, ep):
    Bc, L, _ = XcaP.shape
    NBLK = L // BLKD
    xr = jnp.concatenate([XcaP, ridxP], axis=-1)
    ptab = _pos_emb_table()
    params = [ep["Wh"], ep["Ws"], ep["bs"].reshape(1, -1),
              ep["Wv"], ep["Wg"], ep["bg"].reshape(1, -1)]
    pspecs = [pl.BlockSpec(w.shape, lambda b, i: (0, 0)) for w in params]
    out = pl.pallas_call(
        _edge_kernel(L),
        grid=(Bc, NBLK),
        in_specs=[
            pl.BlockSpec((1, 1, BLKE, 1), lambda b, i: (b, i, 0, 0)),
            pl.BlockSpec((1, L, 4), lambda b, i: (b, 0, 0)),
            pl.BlockSpec(ptab.shape, lambda b, i: (0, 0)),
        ] + pspecs,
        out_specs=[
            pl.BlockSpec((1, BLKE, ES), lambda b, i: (b, i, 0)),
            pl.BlockSpec((1, BLKE, 3), lambda b, i: (b, i, 0)),
        ],
        out_shape=[
            jax.ShapeDtypeStruct((Bc, L * K, ES), jnp.float32),
            jax.ShapeDtypeStruct((Bc, L * K, 3), jnp.float32),
        ],
    )(srcP, xr, ptab, *params)
    return out[0], out[1]


def _node_kernel(BLKN):
    def body(s_ref, v_ref, Wh, Ws, bs, Wv, Wg, bg, out_ref):
        s = s_ref[0]
        vx, vy, vz = v_ref[0, 0], v_ref[0, 1], v_ref[0, 2]
        p = (Wh, Ws, bs, Wv, Wg, bg)
        so, ux, uy, uz = _gvp_cm(p, s, vx, vy, vz, True)
        so, ux, uy, uz = _ln_cm(so, ux, uy, uz)
        out_ref[0] = jnp.concatenate([so, ux, uy, uz], axis=-1)

    return body


def _node_embed(s_n, V_cm, npar):
    Bc, L, si = s_n.shape
    BLKN = min(512, L)
    NBLK = L // BLKN
    NF = NS + 3 * NV
    params = [npar["Wh"], npar["Ws"], npar["bs"].reshape(1, -1),
              npar["Wv"], npar["Wg"], npar["bg"].reshape(1, -1)]
    pspecs = [pl.BlockSpec(w.shape, lambda b, i: (0, 0)) for w in params]
    out = pl.pallas_call(
        _node_kernel(BLKN),
        grid=(Bc, NBLK),
        in_specs=[
            pl.BlockSpec((1, BLKN, si), lambda b, i: (b, i, 0)),
            pl.BlockSpec((1, 3, BLKN, 3), lambda b, i: (b, 0, i, 0)),
        ] + pspecs,
        out_specs=pl.BlockSpec((1, BLKN, NF), lambda b, i: (b, i, 0)),
        out_shape=jax.ShapeDtypeStruct((Bc, L, NF), jnp.float32),
    )(s_n, V_cm, *params)
    return out


def _conv_layer_kernel(L):
    NF = NS + 3 * NV

    def body(src_ref, nf_ref, se_ref, ve_ref, *refs):
        out_ref = refs[-1]
        prefs = refs[:-1]
        p = [prefs[6 * j:6 * j + 6] for j in range(5)]
        i = pl.program_id(1)
        off = i * BLKD

        onehot = _onehot_of(src_ref, L)

        g = _gdot(onehot, nf_ref[0], terms=2)
        nf_d = nf_ref[0, pl.ds(off, BLKD), :]
        ex = _expand(nf_d)

        se_b = se_ref[0]
        ve = ve_ref[0]
        ms = jnp.concatenate([ex[:, :NS], se_b, g[:, :NS]], axis=-1)
        c0, c1, c2, c3 = NS, NS + NV, NS + 2 * NV, NF
        mvx = jnp.concatenate([ex[:, c0:c1], ve[:, 0:1], g[:, c0:c1]], axis=-1)
        mvy = jnp.concatenate([ex[:, c1:c2], ve[:, 1:2], g[:, c1:c2]], axis=-1)
        mvz = jnp.concatenate([ex[:, c2:c3], ve[:, 2:3], g[:, c2:c3]], axis=-1)

        ms, mvx, mvy, mvz = _gvp_cm(p[0], ms, mvx, mvy, mvz, True)
        ms, mvx, mvy, mvz = _gvp_cm(p[1], ms, mvx, mvy, mvz, True)
        ms, mvx, mvy, mvz = _gvp_cm(p[2], ms, mvx, mvy, mvz, False)

        packed_m = jnp.concatenate([ms, mvx, mvy, mvz], axis=-1)
        aggp = jnp.sum(packed_m.reshape(BLKD, K, NF),
                       axis=1) * np.float32(1.0 / K)
        new = nf_d + aggp

        s1, vx1, vy1, vz1 = _ln_cm(new[:, :NS], new[:, c0:c1],
                                   new[:, c1:c2], new[:, c2:c3])
        hs, hvx, hvy, hvz = _gvp_cm(p[3], s1, vx1, vy1, vz1, True)
        hs, hvx, hvy, hvz = _gvp_cm(p[4], hs, hvx, hvy, hvz, False)
        s2, vx2, vy2, vz2 = _ln_cm(s1 + hs, vx1 + hvx, vy1 + hvy, vz1 + hvz)

        out_ref[0] = jnp.concatenate([s2, vx2, vy2, vz2], axis=-1)

    return body


def _conv_layer(nf, seP, veP, srcP, lp):
    Bc, L, NF = nf.shape
    NBLK = L // BLKD
    params = []
    for name in ("msg0", "msg1", "msg2", "ff0", "ff1"):
        g = lp[name]
        params += [g["Wh"], g["Ws"], g["bs"].reshape(1, -1),
                   g["Wv"], g["Wg"], g["bg"].reshape(1, -1)]
    pspecs = [pl.BlockSpec(w.shape, lambda b, i: (0, 0)) for w in params]
    out = pl.pallas_call(
        _conv_layer_kernel(L),
        grid=(Bc, NBLK),
        in_specs=[
            pl.BlockSpec((1, 1, BLKE, 1), lambda b, i: (b, i, 0, 0)),
            pl.BlockSpec((1, L, NF), lambda b, i: (b, 0, 0)),
            pl.BlockSpec((1, BLKE, ES), lambda b, i: (b, i, 0)),
            pl.BlockSpec((1, BLKE, 3), lambda b, i: (b, i, 0)),
        ] + pspecs,
        out_specs=pl.BlockSpec((1, BLKD, NF), lambda b, i: (b, i, 0)),
        out_shape=jax.ShapeDtypeStruct((Bc, L, NF), jnp.float32),
    )(srcP, nf, seP, veP, *params)
    return out


def kernel(coords, coord_mask, res_idx, padding_mask, confidence, params):
    Bc, L = coords.shape[0], coords.shape[1]
    Xca = coords[:, :, 1]

    nbr = _knn(Xca, padding_mask)
    srcP = nbr.reshape(Bc, L // BLKD, BLKD * K, 1)

    ridxP = res_idx.astype(jnp.float32)[..., None]
    seP, veP = _edge_features(Xca, ridxP, srcP, params["edge_embed"])

    s_n = jnp.concatenate([_dihedrals(coords), confidence[..., None]], axis=-1)
    V_n = jnp.concatenate([_orientations(Xca), _sidechains(coords)], axis=-2)
    V_cm = V_n.reshape(Bc, L, 3, 3).transpose(0, 3, 1, 2)
    nf = _node_embed(s_n, V_cm, params["node_embed"])

    for lp in params["layers"]:
        nf = _conv_layer(nf, seP, veP, srcP, lp)

    s_out = nf[:, :, :NS]
    V_out = nf[:, :, NS:].reshape(Bc, L, 3, NV).transpose(0, 1, 3, 2)
    return s_out, V_out
